# baseline (speedup 1.0000x reference)
import functools
import numpy as np
import jax
import jax.numpy as jnp
from jax import lax
from jax.experimental import pallas as pl
from jax.experimental.pallas import tpu as pltpu
from jax.experimental.pallas import tpu_sc as plsc

_N = 10000
_E = 320000
_F = 128
_B = 128
_AVG_DEG_LOG = float(np.log(33.0))
_BIG = 1e30

_K = 128
_RN = 160
_NRANGE = 64
_NP = _RN * _NRANGE
_EPAD = _E + 2 * _K + 32


def _sc_seg_body(bm_hbm, cs_hbm, src_hbm, dst_hbm, eid_hbm, bnd_hbm,
                 s1_hbm, smin_hbm, smax_hbm, s2_hbm,
                 acc1, accmin, accmax, acc2,
                 idx_v, eidv, dstv, brow, crow, bnd_v, sem):
    w = lax.axis_index("s") * 2 + lax.axis_index("c")
    pltpu.sync_copy(bnd_hbm, bnd_v.at[pl.ds(0, 2 * _NRANGE)])
    for p in range(2):
        r = w * 2 + p
        n0 = r * _RN
        pos = 4 * w + 2 * p
        e0 = bnd_v[pl.ds(pos, 16)][0]
        e1 = bnd_v[pl.ds(pos + 1, 16)][0]

        zero16 = jnp.zeros((16,), jnp.float32)

        def zbody(i, _):
            for c in range(8):
                sl = pl.ds(c * 16, 16)
                acc1[i, sl] = zero16
                accmin[i, sl] = zero16 + 3e38
                accmax[i, sl] = zero16 - 3e38
                acc2[i, sl] = zero16
            return 0
        lax.fori_loop(0, _RN, zbody, 0, unroll=False)

        base0 = (e0 // 8) * 8
        nchunk = (e1 - base0 + _K - 1) // _K

        def chunk_body(k, _):
            cstart = base0 + k * _K
            pltpu.sync_copy(src_hbm.at[pl.ds(cstart, _K)], idx_v)
            pltpu.sync_copy(eid_hbm.at[pl.ds(cstart, _K)], eidv)
            pltpu.sync_copy(dst_hbm.at[pl.ds(cstart, _K)],
                            dstv.at[pl.ds(0, _K)])
            cp1 = pltpu.async_copy(bm_hbm.at[idx_v], brow, sem)
            cp2 = pltpu.async_copy(cs_hbm.at[eidv], crow, sem)
            cp1.wait()
            cp2.wait()
            lo = jnp.maximum(e0 - cstart, 0)
            hi = jnp.minimum(e1 - cstart, _K)

            def edge_body(e, _):
                d = dstv[pl.ds(e, 16)][0] - n0
                for c in range(8):
                    sl = pl.ds(c * 16, 16)
                    u = brow[e, sl] + crow[e, sl]
                    acc1[d, sl] += u
                    accmin[d, sl] = jnp.minimum(accmin[d, sl], u)
                    accmax[d, sl] = jnp.maximum(accmax[d, sl], u)
                    acc2[d, sl] += u * u
                return 0
            lax.fori_loop(lo, hi, edge_body, 0, unroll=False)
            return 0
        lax.fori_loop(0, nchunk, chunk_body, 0, unroll=False)

        pltpu.sync_copy(acc1, s1_hbm.at[pl.ds(n0, _RN), :])
        pltpu.sync_copy(accmin, smin_hbm.at[pl.ds(n0, _RN), :])
        pltpu.sync_copy(accmax, smax_hbm.at[pl.ds(n0, _RN), :])
        pltpu.sync_copy(acc2, s2_hbm.at[pl.ds(n0, _RN), :])


def _make_sc_seg_call():
    f32 = jnp.float32
    i32 = jnp.int32
    mesh = plsc.VectorSubcoreMesh(core_axis_name="c", subcore_axis_name="s")
    return pl.kernel(
        _sc_seg_body,
        out_type=[jax.ShapeDtypeStruct((_NP, _F), f32) for _ in range(4)],
        mesh=mesh,
        scratch_types=[
            pltpu.VMEM((_RN, _F), f32), pltpu.VMEM((_RN, _F), f32),
            pltpu.VMEM((_RN, _F), f32), pltpu.VMEM((_RN, _F), f32),
            pltpu.VMEM((_K,), i32), pltpu.VMEM((_K,), i32),
            pltpu.VMEM((_K + 16,), i32),
            pltpu.VMEM((_K, _F), f32), pltpu.VMEM((_K, _F), f32),
            pltpu.VMEM((2 * _NRANGE + 16,), i32),
            pltpu.SemaphoreType.DMA,
        ],
    )


_CBLK = 4000


def _cmat_body(attr_ref, we_ref, wpe_ref, be_ref, c_ref):
    wfold = we_ref[...] @ wpe_ref[...]
    bfold = be_ref[...] @ wpe_ref[...]
    c_ref[...] = attr_ref[...] @ wfold + bfold


def _make_cmat_call():
    f32 = jnp.float32
    return pl.pallas_call(
        _cmat_body,
        grid=(_E // _CBLK,),
        in_specs=[
            pl.BlockSpec((_CBLK, 8), lambda i: (i, 0)),
            pl.BlockSpec((8, _F), lambda i: (0, 0)),
            pl.BlockSpec((_F, _F), lambda i: (0, 0)),
            pl.BlockSpec((1, _F), lambda i: (0, 0)),
        ],
        out_specs=pl.BlockSpec((_CBLK, _F), lambda i: (i, 0)),
        out_shape=jax.ShapeDtypeStruct((_E, _F), f32),
    )


_R = 2000


def _pre_body(x_ref, w1_ref, b1_ref, wd_ref, ws_ref, bpre_ref,
              h_ref, a_ref, bm_ref):
    h = jnp.maximum(x_ref[...] @ w1_ref[...] + b1_ref[...], 0.0)
    h_ref[...] = h
    a_ref[...] = h @ wd_ref[...] + bpre_ref[...]
    bm_ref[...] = h @ ws_ref[...]


def _ab_body(z_ref, s_ref, t_ref, wd_ref, ws_ref, bpre_ref,
             h_ref, a_ref, bm_ref):
    h = jnp.maximum(z_ref[...] * s_ref[...] + t_ref[...], 0.0)
    h_ref[...] = h
    a_ref[...] = h @ wd_ref[...] + bpre_ref[...]
    bm_ref[...] = h @ ws_ref[...]


def _post_body(h_ref, a_ref, s1_ref, smin_ref, smax_ref, s2_ref, deg_ref,
               wh_ref, wa_ref, wamp_ref, watt_ref, bpost_ref, wlin_ref,
               blin_ref, z_ref, cs_ref, csq_ref):
    deg = deg_ref[...]
    degc = jnp.maximum(deg, 1.0)
    a = a_ref[...]
    s1 = s1_ref[...]
    inv = 1.0 / degc
    has = deg > 0.0
    mean = a * (deg * inv) + s1 * inv
    mn = jnp.where(has, a + smin_ref[...], 0.0)
    mx = jnp.where(has, a + smax_ref[...], 0.0)
    msq = (deg * a * a + 2.0 * a * s1 + s2_ref[...]) * inv
    var = msq - mean * mean
    std = jnp.sqrt(jnp.maximum(var, 0.0) + 1e-5)
    agg = jnp.concatenate([mean, mn, mx, std], axis=-1)
    lg = jnp.log(degc + 1.0)
    amp = lg * (1.0 / _AVG_DEG_LOG)
    att = _AVG_DEG_LOG / lg
    y = (h_ref[...] @ wh_ref[...] + agg @ wa_ref[...]
         + amp * (agg @ wamp_ref[...]) + att * (agg @ watt_ref[...])
         + bpost_ref[...])
    z = y @ wlin_ref[...] + blin_ref[...]
    z_ref[...] = z

    @pl.when(pl.program_id(0) == 0)
    def _init():
        cs_ref[...] = jnp.zeros_like(cs_ref)
        csq_ref[...] = jnp.zeros_like(csq_ref)

    cs_ref[...] += jnp.sum(z, axis=0, keepdims=True)
    csq_ref[...] += jnp.sum(z * z, axis=0, keepdims=True)


def _s2s_body(x_ref, bt_ref, s_aff_ref, t_aff_ref, wih_ref, whh_ref, bih_ref,
              bhh_ref, w1_ref, wt_ref, wp_ref, b1_ref, w2_ref, b2_ref,
              w3_ref, b3_ref, tt_ref, pp_ref, out_ref):
    x = jnp.maximum(x_ref[...] * s_aff_ref[...] + t_aff_ref[...], 0.0)
    bt = bt_ref[...]
    cols = jax.lax.broadcasted_iota(jnp.int32, (_N, _B), 1)
    sel = (bt == cols).astype(jnp.float32)
    h = jnp.zeros((_B, _F), jnp.float32)
    c = jnp.zeros((_B, _F), jnp.float32)
    q_star = jnp.zeros((_B, 2 * _F), jnp.float32)
    for _ in range(3):
        gates = (q_star @ wih_ref[...] + bih_ref[...]
                 + h @ whh_ref[...] + bhh_ref[...])
        i_g, f_g, g_g, o_g = jnp.split(gates, 4, axis=-1)
        c = jax.nn.sigmoid(f_g) * c + jax.nn.sigmoid(i_g) * jnp.tanh(g_g)
        h = jax.nn.sigmoid(o_g) * jnp.tanh(c)
        q = h
        qn = sel @ q
        e = jnp.sum(x * qn, axis=-1, keepdims=True)
        emax = jnp.max(sel * e + (sel - 1.0) * _BIG, axis=0, keepdims=True)
        emax = jnp.where(emax > -_BIG * 0.5, emax, 0.0)
        a = jnp.exp(e - jnp.sum(sel * emax, axis=-1, keepdims=True))
        denom = jnp.sum(sel * a, axis=0, keepdims=True)
        denom_n = jnp.sum(sel * denom, axis=-1, keepdims=True)
        a = a / jnp.maximum(denom_n, 1e-16)
        r = jax.lax.dot_general(sel * a, x, (((0,), (0,)), ((), ())))
        q_star = jnp.concatenate([q, r], axis=-1)
    o1 = jnp.maximum(q_star @ w1_ref[...] + tt_ref[...] @ wt_ref[...]
                     + pp_ref[...] @ wp_ref[...] + b1_ref[...], 0.0)
    o2 = jnp.maximum(o1 @ w2_ref[...] + b2_ref[...], 0.0)
    out_ref[...] = o2 @ w3_ref[...] + b3_ref[...]


def _row_bs(shape):
    return pl.BlockSpec((_R,) + shape[1:], lambda i: (0,) * len(shape))


def _full_bs(shape):
    return pl.BlockSpec(shape, lambda i: (0,) * len(shape))


def _node_call(body, in_shapes, n_row_in, n_row_out, extra_out=()):
    grid = _N // _R
    in_specs = []
    for k, s in enumerate(in_shapes):
        if k < n_row_in:
            in_specs.append(pl.BlockSpec(
                (_R,) + tuple(s[1:]),
                lambda i, r=len(s): (i,) + (0,) * (r - 1)))
        else:
            in_specs.append(pl.BlockSpec(
                tuple(s), lambda i, r=len(s): (0,) * r))
    out_specs = [pl.BlockSpec((_R, _F), lambda i: (i, 0))
                 for _ in range(n_row_out)]
    out_shape = [jax.ShapeDtypeStruct((_N, _F), jnp.float32)
                 for _ in range(n_row_out)]
    for s in extra_out:
        out_specs.append(pl.BlockSpec(s, lambda i: (0,) * len(s)))
        out_shape.append(jax.ShapeDtypeStruct(s, jnp.float32))
    return pl.pallas_call(body, grid=(grid,), in_specs=in_specs,
                          out_specs=out_specs, out_shape=out_shape)


def _vmem_call(body, out_shape):
    return pl.pallas_call(body, out_shape=out_shape)


def kernel(x, edge_index, edge_attr, batch, t, p, params):
    src = edge_index[0]
    dst = edge_index[1]
    i32 = jnp.int32

    eids = jnp.arange(_E, dtype=i32)
    dst_sorted, perm = jax.lax.sort_key_val(dst, eids)
    offsets = jnp.searchsorted(
        dst_sorted, jnp.arange(_N + 1, dtype=i32), method='sort').astype(i32)
    deg = (offsets[1:] - offsets[:-1]).astype(jnp.float32).reshape(_N, 1)
    src_sorted = src[perm]

    pad1 = jnp.zeros((_EPAD - _E,), i32)
    src_pad = jnp.concatenate([src_sorted, pad1])
    dst_pad = jnp.concatenate([dst_sorted, pad1])
    eid_pad = jnp.concatenate([perm, pad1])
    rb = jnp.minimum(jnp.arange(_NRANGE + 1, dtype=i32) * _RN, _N)
    b65 = offsets[rb]
    bounds = jnp.stack([b65[:_NRANGE], b65[1:]], axis=1).reshape(-1)

    attr8 = jnp.concatenate(
        [edge_attr, jnp.zeros((_E, 8 - edge_attr.shape[1]), jnp.float32)],
        axis=1)
    cmat = _make_cmat_call()
    seg = _make_sc_seg_call()

    W1, b1 = params['lin1']
    convs = params['convs']

    f32 = jnp.float32
    pre = _node_call(
        _pre_body,
        [(_N, _F), (_F, _F), (1, _F), (_F, _F), (_F, _F), (1, _F)],
        n_row_in=1, n_row_out=3)
    ab = _node_call(
        _ab_body,
        [(_N, _F), (1, _F), (1, _F), (_F, _F), (_F, _F), (1, _F)],
        n_row_in=1, n_row_out=3)
    post = _node_call(
        _post_body,
        [(_N, _F)] * 6 + [(_N, 1)] + [(_F, _F), (4 * _F, _F), (4 * _F, _F),
                                      (4 * _F, _F), (1, _F), (_F, _F),
                                      (1, _F)],
        n_row_in=7, n_row_out=1, extra_out=[(1, _F), (1, _F)])

    h = a = bm = None
    z = cs = csq = None
    for li, cp in enumerate(convs):
        wd = cp['Wpre'][:_F]
        ws = cp['Wpre'][_F:2 * _F]
        wpe = cp['Wpre'][2 * _F:]
        bpre = cp['bpre'].reshape(1, _F)
        if li == 0:
            h, a, bm = pre(x, W1, b1.reshape(1, _F), wd, ws, bpre)
        else:
            mean_c = cs / _N
            var_c = csq / _N - mean_c * mean_c
            s_aff = cp_prev['gamma'].reshape(1, _F) / jnp.sqrt(var_c + 1e-5)
            t_aff = cp_prev['beta'].reshape(1, _F) - mean_c * s_aff
            h, a, bm = ab(z, s_aff, t_aff, wd, ws, bpre)
        we8 = jnp.concatenate(
            [cp['We'], jnp.zeros((2, _F), jnp.float32)], axis=0)
        c_edges = cmat(attr8, we8, wpe, cp['be'].reshape(1, _F))
        s1p, sminp, smaxp, s2p = seg(bm, c_edges, src_pad, dst_pad,
                                     eid_pad, bounds)
        s1, smin, smax, s2 = (s1p[:_N], sminp[:_N], smaxp[:_N], s2p[:_N])
        wh = cp['Wpost'][:_F]
        wa = cp['Wpost'][_F:5 * _F]
        wamp = cp['Wpost'][5 * _F:9 * _F]
        watt = cp['Wpost'][9 * _F:]
        z, cs, csq = post(h, a, s1, smin, smax, s2, deg, wh, wa, wamp, watt,
                          cp['bpost'].reshape(1, _F), cp['Wlin'],
                          cp['blin'].reshape(1, _F))
        cp_prev = cp

    mean_c = cs / _N
    var_c = csq / _N - mean_c * mean_c
    s_aff = cp_prev['gamma'].reshape(1, _F) / jnp.sqrt(var_c + 1e-5)
    t_aff = cp_prev['beta'].reshape(1, _F) - mean_c * s_aff

    lstm = params['lstm']
    (W1m, b1m), (W2m, b2m), (W3m, b3m) = params['mlp']
    s2s = _vmem_call(_s2s_body, jax.ShapeDtypeStruct((_B, 1), f32))
    out = s2s(z, batch.reshape(_N, 1), s_aff, t_aff,
              lstm['W_ih'].T, lstm['W_hh'].T,
              lstm['b_ih'].reshape(1, 4 * _F), lstm['b_hh'].reshape(1, 4 * _F),
              W1m[:2 * _F], W1m[2 * _F:2 * _F + 1], W1m[2 * _F + 1:],
              b1m.reshape(1, 64), W2m, b2m.reshape(1, 32), W3m,
              b3m.reshape(1, 1), t, p)
    return out.reshape(-1)


# speedup vs baseline: 1.5181x; 1.5181x over previous
import functools
import numpy as np
import jax
import jax.numpy as jnp
from jax import lax
from jax.experimental import pallas as pl
from jax.experimental.pallas import tpu as pltpu
from jax.experimental.pallas import tpu_sc as plsc

_N = 10000
_E = 320000
_F = 128
_B = 128
_AVG_DEG_LOG = float(np.log(33.0))
_BIG = 1e30

_K = 80
_SUP = 8
_SUPE = _K * _SUP
_RN = 160
_NRANGE = 64
_NP = _RN * _NRANGE
_EPAD = _E + 1024


def _sc_seg_body(bm_hbm, cs_hbm, src_hbm, eid_hbm, dst_hbm, bnd_hbm,
             s1_hbm, smin_hbm, smax_hbm, s2_hbm,
             acc1, accmin, accmax, acc2,
             sidx, eidx, dstv, browA, crowA, browB, crowB, bnd_v,
             semA, semB):
    w = lax.axis_index("s") * 2 + lax.axis_index("c")
    pltpu.sync_copy(bnd_hbm, bnd_v.at[pl.ds(0, 2 * _NRANGE)])
    zero16 = jnp.zeros((16,), jnp.float32)

    def init_regs():
        return tuple([zero16] * 8 + [zero16 + 3e38] * 8
                     + [zero16 - 3e38] * 8 + [zero16] * 8)

    for p in range(2):
        r = w * 2 + p
        n0 = r * _RN
        pos = 4 * w + 2 * p
        e0 = bnd_v[pl.ds(pos, 16)][0]
        e1 = bnd_v[pl.ds(pos + 1, 16)][0]
        base0 = (e0 // 8) * 8
        nsuper = (e1 - base0 + _SUPE - 1) // _SUPE

        def zbody(i, _):
            for c in range(8):
                sl = pl.ds(i * _F + c * 16, 16)
                acc1[sl] = zero16
                accmin[sl] = zero16
                accmax[sl] = zero16
                acc2[sl] = zero16
            return 0
        lax.fori_loop(0, _RN, zbody, 0, unroll=False)

        def flush(d, regs):
            for c in range(8):
                sl = pl.ds(d * _F + c * 16, 16)
                acc1[sl] = regs[c]
                accmin[sl] = regs[8 + c]
                accmax[sl] = regs[16 + c]
                acc2[sl] = regs[24 + c]

        bufs = ((browA, crowA, semA), (browB, crowB, semB))

        def gathers(c, buf):
            brow, crow, sem = buf
            cp1 = pltpu.async_copy(bm_hbm.at[sidx.at[pl.ds(c * _K, _K)]],
                                   brow, sem)
            cp2 = pltpu.async_copy(cs_hbm.at[eidx.at[pl.ds(c * _K, _K)]],
                                   crow, sem)
            return cp1, cp2

        def consume(carry, brow, crow, sstart, c):
            cstart = sstart + c * _K
            lo = jnp.maximum(e0 - cstart, 0)
            hi = jnp.minimum(e1 - cstart, _K)

            def edge_body(e, st):
                d_prev = st[0]
                d = dstv[pl.ds(c * _K + e, 16)][0] - n0
                pred = d != d_prev

                @pl.when(jnp.logical_and(pred, d_prev >= 0))
                def _():
                    flush(d_prev, st[1:])

                new = [d]
                for cc in range(8):
                    sl = pl.ds(cc * 16, 16)
                    u = brow[e, sl] + crow[e, sl]
                    u2 = u * u
                    new.append(jnp.where(pred, u, st[1 + cc] + u))
                    new.append(jnp.where(pred, u, jnp.minimum(st[9 + cc], u)))
                    new.append(jnp.where(pred, u, jnp.maximum(st[17 + cc], u)))
                    new.append(jnp.where(pred, u2, st[25 + cc] + u2))
                out = [new[0]]
                out += [new[1 + 4 * cc] for cc in range(8)]
                out += [new[2 + 4 * cc] for cc in range(8)]
                out += [new[3 + 4 * cc] for cc in range(8)]
                out += [new[4 + 4 * cc] for cc in range(8)]
                return tuple(out)
            return lax.fori_loop(lo, hi, edge_body, carry, unroll=False)

        def super_body(s, carry):
            sstart = base0 + s * _SUPE
            pltpu.sync_copy(src_hbm.at[pl.ds(sstart, _SUPE)], sidx)
            pltpu.sync_copy(eid_hbm.at[pl.ds(sstart, _SUPE)], eidx)
            pltpu.sync_copy(dst_hbm.at[pl.ds(sstart, _SUPE)],
                            dstv.at[pl.ds(0, _SUPE)])
            pend = gathers(0, bufs[0])
            for c in range(_SUP):
                nxt = gathers(c + 1, bufs[(c + 1) % 2]) if c + 1 < _SUP else None
                pend[0].wait()
                pend[1].wait()
                brow, crow, _ = bufs[c % 2]
                carry = consume(carry, brow, crow, sstart, c)
                pend = nxt
            return carry

        carry = (jnp.int32(-1),) + init_regs()
        carry = lax.fori_loop(0, nsuper, super_body, carry, unroll=False)

        d_last = carry[0]

        @pl.when(d_last >= 0)
        def _():
            flush(d_last, carry[1:])

        pltpu.sync_copy(acc1, s1_hbm.at[pl.ds(n0 * _F, _RN * _F)])
        pltpu.sync_copy(accmin, smin_hbm.at[pl.ds(n0 * _F, _RN * _F)])
        pltpu.sync_copy(accmax, smax_hbm.at[pl.ds(n0 * _F, _RN * _F)])
        pltpu.sync_copy(acc2, s2_hbm.at[pl.ds(n0 * _F, _RN * _F)])


def _make_sc_seg_call():
    f32 = jnp.float32
    i32 = jnp.int32
    mesh = plsc.VectorSubcoreMesh(core_axis_name="c", subcore_axis_name="s")
    return pl.kernel(
        _sc_seg_body,
        out_type=[jax.ShapeDtypeStruct((_NP * _F,), f32) for _ in range(4)],
        mesh=mesh,
        scratch_types=[
            pltpu.VMEM((_RN * _F,), f32), pltpu.VMEM((_RN * _F,), f32),
            pltpu.VMEM((_RN * _F,), f32), pltpu.VMEM((_RN * _F,), f32),
            pltpu.VMEM((_SUPE,), i32), pltpu.VMEM((_SUPE,), i32),
            pltpu.VMEM((_SUPE + 16,), i32),
            pltpu.VMEM((_K, _F), f32), pltpu.VMEM((_K, _F), f32),
            pltpu.VMEM((_K, _F), f32), pltpu.VMEM((_K, _F), f32),
            pltpu.VMEM((2 * _NRANGE + 16,), i32),
            pltpu.SemaphoreType.DMA, pltpu.SemaphoreType.DMA,
        ],
    )


_CBLK = 4000


def _cmat_body(attr_ref, we_ref, wpe_ref, be_ref, c_ref):
    wfold = we_ref[...] @ wpe_ref[...]
    bfold = be_ref[...] @ wpe_ref[...]
    c_ref[...] = attr_ref[...] @ wfold + bfold


def _make_cmat_call():
    f32 = jnp.float32
    return pl.pallas_call(
        _cmat_body,
        grid=(_E // _CBLK,),
        in_specs=[
            pl.BlockSpec((_CBLK, 8), lambda i: (i, 0)),
            pl.BlockSpec((8, _F), lambda i: (0, 0)),
            pl.BlockSpec((_F, _F), lambda i: (0, 0)),
            pl.BlockSpec((1, _F), lambda i: (0, 0)),
        ],
        out_specs=pl.BlockSpec((_CBLK, _F), lambda i: (i, 0)),
        out_shape=jax.ShapeDtypeStruct((_E, _F), f32),
    )


_R = 2000


def _pre_body(x_ref, w1_ref, b1_ref, wd_ref, ws_ref, bpre_ref,
              h_ref, a_ref, bm_ref):
    h = jnp.maximum(x_ref[...] @ w1_ref[...] + b1_ref[...], 0.0)
    h_ref[...] = h
    a_ref[...] = h @ wd_ref[...] + bpre_ref[...]
    bm_ref[...] = h @ ws_ref[...]


def _ab_body(z_ref, s_ref, t_ref, wd_ref, ws_ref, bpre_ref,
             h_ref, a_ref, bm_ref):
    h = jnp.maximum(z_ref[...] * s_ref[...] + t_ref[...], 0.0)
    h_ref[...] = h
    a_ref[...] = h @ wd_ref[...] + bpre_ref[...]
    bm_ref[...] = h @ ws_ref[...]


def _post_body(h_ref, a_ref, s1_ref, smin_ref, smax_ref, s2_ref, deg_ref,
               wh_ref, wa_ref, wamp_ref, watt_ref, bpost_ref, wlin_ref,
               blin_ref, z_ref, cs_ref, csq_ref):
    deg = deg_ref[...]
    degc = jnp.maximum(deg, 1.0)
    a = a_ref[...]
    s1 = s1_ref[...]
    inv = 1.0 / degc
    has = deg > 0.0
    mu = s1 * inv
    mean = a * (deg * inv) + mu
    mn = jnp.where(has, a + smin_ref[...], 0.0)
    mx = jnp.where(has, a + smax_ref[...], 0.0)
    var = s2_ref[...] * inv - mu * mu
    std = jnp.sqrt(jnp.maximum(var, 0.0) + 1e-5)
    agg = jnp.concatenate([mean, mn, mx, std], axis=-1)
    lg = jnp.log(degc + 1.0)
    amp = lg * (1.0 / _AVG_DEG_LOG)
    att = _AVG_DEG_LOG / lg
    y = (h_ref[...] @ wh_ref[...] + agg @ wa_ref[...]
         + amp * (agg @ wamp_ref[...]) + att * (agg @ watt_ref[...])
         + bpost_ref[...])
    z = y @ wlin_ref[...] + blin_ref[...]
    z_ref[...] = z

    @pl.when(pl.program_id(0) == 0)
    def _init():
        cs_ref[...] = jnp.zeros_like(cs_ref)
        csq_ref[...] = jnp.zeros_like(csq_ref)

    cs_ref[...] += jnp.sum(z, axis=0, keepdims=True)
    csq_ref[...] += jnp.sum(z * z, axis=0, keepdims=True)


def _s2s_body(x_ref, bt_ref, s_aff_ref, t_aff_ref, wih_ref, whh_ref, bih_ref,
              bhh_ref, w1_ref, wt_ref, wp_ref, b1_ref, w2_ref, b2_ref,
              w3_ref, b3_ref, tt_ref, pp_ref, out_ref):
    x = jnp.maximum(x_ref[...] * s_aff_ref[...] + t_aff_ref[...], 0.0)
    bt = bt_ref[...]
    cols = jax.lax.broadcasted_iota(jnp.int32, (_N, _B), 1)
    sel = (bt == cols).astype(jnp.float32)
    h = jnp.zeros((_B, _F), jnp.float32)
    c = jnp.zeros((_B, _F), jnp.float32)
    q_star = jnp.zeros((_B, 2 * _F), jnp.float32)
    for _ in range(3):
        gates = (q_star @ wih_ref[...] + bih_ref[...]
                 + h @ whh_ref[...] + bhh_ref[...])
        i_g, f_g, g_g, o_g = jnp.split(gates, 4, axis=-1)
        c = jax.nn.sigmoid(f_g) * c + jax.nn.sigmoid(i_g) * jnp.tanh(g_g)
        h = jax.nn.sigmoid(o_g) * jnp.tanh(c)
        q = h
        qn = sel @ q
        e = jnp.sum(x * qn, axis=-1, keepdims=True)
        emax = jnp.max(sel * e + (sel - 1.0) * _BIG, axis=0, keepdims=True)
        emax = jnp.where(emax > -_BIG * 0.5, emax, 0.0)
        a = jnp.exp(e - jnp.sum(sel * emax, axis=-1, keepdims=True))
        denom = jnp.sum(sel * a, axis=0, keepdims=True)
        denom_n = jnp.sum(sel * denom, axis=-1, keepdims=True)
        a = a / jnp.maximum(denom_n, 1e-16)
        r = jax.lax.dot_general(sel * a, x, (((0,), (0,)), ((), ())))
        q_star = jnp.concatenate([q, r], axis=-1)
    o1 = jnp.maximum(q_star @ w1_ref[...] + tt_ref[...] @ wt_ref[...]
                     + pp_ref[...] @ wp_ref[...] + b1_ref[...], 0.0)
    o2 = jnp.maximum(o1 @ w2_ref[...] + b2_ref[...], 0.0)
    out_ref[...] = o2 @ w3_ref[...] + b3_ref[...]


def _row_bs(shape):
    return pl.BlockSpec((_R,) + shape[1:], lambda i: (0,) * len(shape))


def _full_bs(shape):
    return pl.BlockSpec(shape, lambda i: (0,) * len(shape))


def _node_call(body, in_shapes, n_row_in, n_row_out, extra_out=()):
    grid = _N // _R
    in_specs = []
    for k, s in enumerate(in_shapes):
        if k < n_row_in:
            in_specs.append(pl.BlockSpec(
                (_R,) + tuple(s[1:]),
                lambda i, r=len(s): (i,) + (0,) * (r - 1)))
        else:
            in_specs.append(pl.BlockSpec(
                tuple(s), lambda i, r=len(s): (0,) * r))
    out_specs = [pl.BlockSpec((_R, _F), lambda i: (i, 0))
                 for _ in range(n_row_out)]
    out_shape = [jax.ShapeDtypeStruct((_N, _F), jnp.float32)
                 for _ in range(n_row_out)]
    for s in extra_out:
        out_specs.append(pl.BlockSpec(s, lambda i: (0,) * len(s)))
        out_shape.append(jax.ShapeDtypeStruct(s, jnp.float32))
    return pl.pallas_call(body, grid=(grid,), in_specs=in_specs,
                          out_specs=out_specs, out_shape=out_shape)


def _vmem_call(body, out_shape):
    return pl.pallas_call(body, out_shape=out_shape)


def kernel(x, edge_index, edge_attr, batch, t, p, params):
    src = edge_index[0]
    dst = edge_index[1]
    i32 = jnp.int32

    eids = jnp.arange(_E, dtype=i32)
    dst_sorted, perm = jax.lax.sort_key_val(dst, eids)
    offsets = jnp.searchsorted(
        dst_sorted, jnp.arange(_N + 1, dtype=i32)).astype(i32)
    deg = (offsets[1:] - offsets[:-1]).astype(jnp.float32).reshape(_N, 1)
    src_sorted = src[perm]

    pad1 = jnp.zeros((_EPAD - _E,), i32)
    src_pad = jnp.concatenate([src_sorted, pad1])
    dst_pad = jnp.concatenate([dst_sorted, pad1])
    eid_pad = jnp.concatenate([perm, pad1])
    rb = jnp.minimum(jnp.arange(_NRANGE + 1, dtype=i32) * _RN, _N)
    b65 = offsets[rb]
    bounds = jnp.stack([b65[:_NRANGE], b65[1:]], axis=1).reshape(-1)

    attr8 = jnp.concatenate(
        [edge_attr, jnp.zeros((_E, 8 - edge_attr.shape[1]), jnp.float32)],
        axis=1)
    cmat = _make_cmat_call()
    seg = _make_sc_seg_call()

    W1, b1 = params['lin1']
    convs = params['convs']

    f32 = jnp.float32
    pre = _node_call(
        _pre_body,
        [(_N, _F), (_F, _F), (1, _F), (_F, _F), (_F, _F), (1, _F)],
        n_row_in=1, n_row_out=3)
    ab = _node_call(
        _ab_body,
        [(_N, _F), (1, _F), (1, _F), (_F, _F), (_F, _F), (1, _F)],
        n_row_in=1, n_row_out=3)
    post = _node_call(
        _post_body,
        [(_N, _F)] * 6 + [(_N, 1)] + [(_F, _F), (4 * _F, _F), (4 * _F, _F),
                                      (4 * _F, _F), (1, _F), (_F, _F),
                                      (1, _F)],
        n_row_in=7, n_row_out=1, extra_out=[(1, _F), (1, _F)])

    h = a = bm = None
    z = cs = csq = None
    for li, cp in enumerate(convs):
        wd = cp['Wpre'][:_F]
        ws = cp['Wpre'][_F:2 * _F]
        wpe = cp['Wpre'][2 * _F:]
        bpre = cp['bpre'].reshape(1, _F)
        if li == 0:
            h, a, bm = pre(x, W1, b1.reshape(1, _F), wd, ws, bpre)
        else:
            mean_c = cs / _N
            var_c = csq / _N - mean_c * mean_c
            s_aff = cp_prev['gamma'].reshape(1, _F) / jnp.sqrt(var_c + 1e-5)
            t_aff = cp_prev['beta'].reshape(1, _F) - mean_c * s_aff
            h, a, bm = ab(z, s_aff, t_aff, wd, ws, bpre)
        we8 = jnp.concatenate(
            [cp['We'], jnp.zeros((2, _F), jnp.float32)], axis=0)
        c_edges = cmat(attr8, we8, wpe, cp['be'].reshape(1, _F))
        s1p, sminp, smaxp, s2p = seg(bm, c_edges, src_pad, eid_pad,
                                     dst_pad, bounds)
        s1 = s1p.reshape(_NP, _F)[:_N]
        smin = sminp.reshape(_NP, _F)[:_N]
        smax = smaxp.reshape(_NP, _F)[:_N]
        s2 = s2p.reshape(_NP, _F)[:_N]
        wh = cp['Wpost'][:_F]
        wa = cp['Wpost'][_F:5 * _F]
        wamp = cp['Wpost'][5 * _F:9 * _F]
        watt = cp['Wpost'][9 * _F:]
        z, cs, csq = post(h, a, s1, smin, smax, s2, deg, wh, wa, wamp, watt,
                          cp['bpost'].reshape(1, _F), cp['Wlin'],
                          cp['blin'].reshape(1, _F))
        cp_prev = cp

    mean_c = cs / _N
    var_c = csq / _N - mean_c * mean_c
    s_aff = cp_prev['gamma'].reshape(1, _F) / jnp.sqrt(var_c + 1e-5)
    t_aff = cp_prev['beta'].reshape(1, _F) - mean_c * s_aff

    lstm = params['lstm']
    (W1m, b1m), (W2m, b2m), (W3m, b3m) = params['mlp']
    s2s = _vmem_call(_s2s_body, jax.ShapeDtypeStruct((_B, 1), f32))
    out = s2s(z, batch.reshape(_N, 1), s_aff, t_aff,
              lstm['W_ih'].T, lstm['W_hh'].T,
              lstm['b_ih'].reshape(1, 4 * _F), lstm['b_hh'].reshape(1, 4 * _F),
              W1m[:2 * _F], W1m[2 * _F:2 * _F + 1], W1m[2 * _F + 1:],
              b1m.reshape(1, 64), W2m, b2m.reshape(1, 32), W3m,
              b3m.reshape(1, 1), t, p)
    return out.reshape(-1)


# speedup vs baseline: 1.9028x; 1.2534x over previous
import functools
import numpy as np
import jax
import jax.numpy as jnp
from jax import lax
from jax.experimental import pallas as pl
from jax.experimental.pallas import tpu as pltpu
from jax.experimental.pallas import tpu_sc as plsc

_N = 10000
_E = 320000
_F = 128
_B = 128
_AVG_DEG_LOG = float(np.log(33.0))
_BIG = 1e30

_K = 80
_SUP = 8
_SUPE = _K * _SUP
_RN = 160
_NRANGE = 64
_NP = _RN * _NRANGE
_EPAD = _E + 1024


def _sc_seg_body(bm_hbm, cs_hbm, src_hbm, eid_hbm, dst_hbm,
             s1_hbm, smin_hbm, smax_hbm, s2_hbm, deg_hbm,
             acc1, accmin, accmax, acc2, degacc,
             sidx, eidx, dstv, browA, crowA, browB, crowB, probev,
             semA, semB, semS):
    w = lax.axis_index("s") * 2 + lax.axis_index("c")
    zero16 = jnp.zeros((16,), jnp.float32)

    def init_regs():
        return tuple([zero16] * 8 + [zero16 + 3e38] * 8
                     + [zero16 - 3e38] * 8 + [zero16] * 8 + [zero16])

    def lower_bound(v):
        def it(t, lohi):
            lo, hi = lohi
            mid = (lo + hi) // 2
            m8 = (mid // 8) * 8
            pltpu.sync_copy(dst_hbm.at[pl.ds(m8, 16)],
                            probev.at[pl.ds(0, 16)])
            dv = probev[pl.ds(mid - m8, 16)][0]
            less = dv < v
            return (jnp.where(less, mid + 1, lo), jnp.where(less, hi, mid))
        lo, hi = lax.fori_loop(0, 19, it, (jnp.int32(0), jnp.int32(_E)),
                               unroll=False)
        return lo

    eb0 = lower_bound((w * 2) * _RN)
    eb1 = lower_bound((w * 2 + 1) * _RN)
    eb2 = lower_bound((w * 2 + 2) * _RN)

    for p in range(2):
        r = w * 2 + p
        n0 = r * _RN
        e0 = eb0 if p == 0 else eb1
        e1 = eb1 if p == 0 else eb2
        base0 = (e0 // 8) * 8
        nsuper = (e1 - base0 + _SUPE - 1) // _SUPE

        def zbody(i, _):
            for c in range(8):
                sl = pl.ds(i * _F + c * 16, 16)
                acc1[sl] = zero16
                accmin[sl] = zero16
                accmax[sl] = zero16
                acc2[sl] = zero16
            degacc[pl.ds(i * 16, 16)] = zero16
            return 0
        lax.fori_loop(0, _RN, zbody, 0, unroll=False)

        def flush(d, regs):
            for c in range(8):
                sl = pl.ds(d * _F + c * 16, 16)
                acc1[sl] = regs[c]
                accmin[sl] = regs[8 + c]
                accmax[sl] = regs[16 + c]
                acc2[sl] = regs[24 + c]
            degacc[pl.ds(d * 16, 16)] = regs[32]

        bufs = ((browA, crowA, semA), (browB, crowB, semB))

        def gathers(c, buf):
            brow, crow, sem = buf
            cp1 = pltpu.async_copy(bm_hbm.at[sidx.at[pl.ds(c * _K, _K)]],
                                   brow, sem)
            cp2 = pltpu.async_copy(cs_hbm.at[eidx.at[pl.ds(c * _K, _K)]],
                                   crow, sem)
            return cp1, cp2

        def consume(carry, brow, crow, sstart, c):
            cstart = sstart + c * _K
            lo = jnp.maximum(e0 - cstart, 0)
            hi = jnp.minimum(e1 - cstart, _K)

            def edge_body(e, st):
                d_prev = st[0]
                d = dstv[pl.ds(c * _K + e, 16)][0] - n0
                pred = d != d_prev

                @pl.when(jnp.logical_and(pred, d_prev >= 0))
                def _():
                    flush(d_prev, st[1:])

                new = [d]
                for cc in range(8):
                    sl = pl.ds(cc * 16, 16)
                    u = brow[e, sl] + crow[e, sl]
                    u2 = u * u
                    new.append(jnp.where(pred, u, st[1 + cc] + u))
                    new.append(jnp.where(pred, u, jnp.minimum(st[9 + cc], u)))
                    new.append(jnp.where(pred, u, jnp.maximum(st[17 + cc], u)))
                    new.append(jnp.where(pred, u2, st[25 + cc] + u2))
                out = [new[0]]
                out += [new[1 + 4 * cc] for cc in range(8)]
                out += [new[2 + 4 * cc] for cc in range(8)]
                out += [new[3 + 4 * cc] for cc in range(8)]
                out += [new[4 + 4 * cc] for cc in range(8)]
                out.append(jnp.where(pred, zero16 + 1.0, st[33] + 1.0))
                return tuple(out)
            return lax.fori_loop(lo, hi, edge_body, carry, unroll=False)

        def super_body(s, carry):
            sstart = base0 + s * _SUPE
            pltpu.sync_copy(eid_hbm.at[pl.ds(sstart, _SUPE)], eidx)
            pltpu.sync_copy(dst_hbm.at[pl.ds(sstart, _SUPE)],
                            dstv.at[pl.ds(0, _SUPE)])
            scp = [pltpu.async_copy(
                src_hbm.at[eidx.at[pl.ds(j * _K, _K)]],
                sidx.at[pl.ds(j * _K, _K)], semS) for j in range(_SUP)]
            for cp in scp:
                cp.wait()
            pend = gathers(0, bufs[0])
            for c in range(_SUP):
                nxt = gathers(c + 1, bufs[(c + 1) % 2]) if c + 1 < _SUP else None
                pend[0].wait()
                pend[1].wait()
                brow, crow, _ = bufs[c % 2]
                carry = consume(carry, brow, crow, sstart, c)
                pend = nxt
            return carry

        carry = (jnp.int32(-1),) + init_regs()
        carry = lax.fori_loop(0, nsuper, super_body, carry, unroll=False)

        d_last = carry[0]

        @pl.when(d_last >= 0)
        def _():
            flush(d_last, carry[1:])

        pltpu.sync_copy(acc1, s1_hbm.at[pl.ds(n0 * _F, _RN * _F)])
        pltpu.sync_copy(accmin, smin_hbm.at[pl.ds(n0 * _F, _RN * _F)])
        pltpu.sync_copy(accmax, smax_hbm.at[pl.ds(n0 * _F, _RN * _F)])
        pltpu.sync_copy(acc2, s2_hbm.at[pl.ds(n0 * _F, _RN * _F)])
        pltpu.sync_copy(degacc, deg_hbm.at[pl.ds(n0 * 16, _RN * 16)])


def _make_sc_seg_call():
    f32 = jnp.float32
    i32 = jnp.int32
    mesh = plsc.VectorSubcoreMesh(core_axis_name="c", subcore_axis_name="s")
    return pl.kernel(
        _sc_seg_body,
        out_type=[jax.ShapeDtypeStruct((_NP * _F,), f32) for _ in range(4)]
        + [jax.ShapeDtypeStruct((_NP * 16,), f32)],
        mesh=mesh,
        scratch_types=[
            pltpu.VMEM((_RN * _F,), f32), pltpu.VMEM((_RN * _F,), f32),
            pltpu.VMEM((_RN * _F,), f32), pltpu.VMEM((_RN * _F,), f32),
            pltpu.VMEM((_RN * 16,), f32),
            pltpu.VMEM((_SUPE,), i32), pltpu.VMEM((_SUPE,), i32),
            pltpu.VMEM((_SUPE + 16,), i32),
            pltpu.VMEM((_K, _F), f32), pltpu.VMEM((_K, _F), f32),
            pltpu.VMEM((_K, _F), f32), pltpu.VMEM((_K, _F), f32),
            pltpu.VMEM((32,), i32),
            pltpu.SemaphoreType.DMA, pltpu.SemaphoreType.DMA,
            pltpu.SemaphoreType.DMA,
        ],
    )


_CBLK = 4000


def _cmat_body(attr_ref, we_ref, wpe_ref, be_ref, c_ref):
    wfold = we_ref[...] @ wpe_ref[...]
    bfold = be_ref[...] @ wpe_ref[...]
    c_ref[...] = attr_ref[...] @ wfold + bfold


def _make_cmat_call():
    f32 = jnp.float32
    return pl.pallas_call(
        _cmat_body,
        grid=(_E // _CBLK,),
        in_specs=[
            pl.BlockSpec((_CBLK, 8), lambda i: (i, 0)),
            pl.BlockSpec((8, _F), lambda i: (0, 0)),
            pl.BlockSpec((_F, _F), lambda i: (0, 0)),
            pl.BlockSpec((1, _F), lambda i: (0, 0)),
        ],
        out_specs=pl.BlockSpec((_CBLK, _F), lambda i: (i, 0)),
        out_shape=jax.ShapeDtypeStruct((_E, _F), f32),
    )


_R = 2000


def _pre_body(x_ref, w1_ref, b1_ref, wd_ref, ws_ref, bpre_ref,
              h_ref, a_ref, bm_ref):
    h = jnp.maximum(x_ref[...] @ w1_ref[...] + b1_ref[...], 0.0)
    h_ref[...] = h
    a_ref[...] = h @ wd_ref[...] + bpre_ref[...]
    bm_ref[...] = h @ ws_ref[...]


def _ab_body(z_ref, s_ref, t_ref, wd_ref, ws_ref, bpre_ref,
             h_ref, a_ref, bm_ref):
    h = jnp.maximum(z_ref[...] * s_ref[...] + t_ref[...], 0.0)
    h_ref[...] = h
    a_ref[...] = h @ wd_ref[...] + bpre_ref[...]
    bm_ref[...] = h @ ws_ref[...]


def _post_body(h_ref, a_ref, s1_ref, smin_ref, smax_ref, s2_ref, deg_ref,
               wh_ref, wa_ref, wamp_ref, watt_ref, bpost_ref, wlin_ref,
               blin_ref, z_ref, cs_ref, csq_ref):
    deg = deg_ref[...]
    degc = jnp.maximum(deg, 1.0)
    a = a_ref[...]
    s1 = s1_ref[...]
    inv = 1.0 / degc
    has = deg > 0.0
    mu = s1 * inv
    mean = a * (deg * inv) + mu
    mn = jnp.where(has, a + smin_ref[...], 0.0)
    mx = jnp.where(has, a + smax_ref[...], 0.0)
    var = s2_ref[...] * inv - mu * mu
    std = jnp.sqrt(jnp.maximum(var, 0.0) + 1e-5)
    agg = jnp.concatenate([mean, mn, mx, std], axis=-1)
    lg = jnp.log(degc + 1.0)
    amp = lg * (1.0 / _AVG_DEG_LOG)
    att = _AVG_DEG_LOG / lg
    y = (h_ref[...] @ wh_ref[...] + agg @ wa_ref[...]
         + amp * (agg @ wamp_ref[...]) + att * (agg @ watt_ref[...])
         + bpost_ref[...])
    z = y @ wlin_ref[...] + blin_ref[...]
    z_ref[...] = z

    @pl.when(pl.program_id(0) == 0)
    def _init():
        cs_ref[...] = jnp.zeros_like(cs_ref)
        csq_ref[...] = jnp.zeros_like(csq_ref)

    cs_ref[...] += jnp.sum(z, axis=0, keepdims=True)
    csq_ref[...] += jnp.sum(z * z, axis=0, keepdims=True)


def _s2s_body(x_ref, bt_ref, s_aff_ref, t_aff_ref, wih_ref, whh_ref, bih_ref,
              bhh_ref, w1_ref, wt_ref, wp_ref, b1_ref, w2_ref, b2_ref,
              w3_ref, b3_ref, tt_ref, pp_ref, out_ref):
    x = jnp.maximum(x_ref[...] * s_aff_ref[...] + t_aff_ref[...], 0.0)
    bt = bt_ref[...]
    cols = jax.lax.broadcasted_iota(jnp.int32, (_N, _B), 1)
    sel = (bt == cols).astype(jnp.float32)
    h = jnp.zeros((_B, _F), jnp.float32)
    c = jnp.zeros((_B, _F), jnp.float32)
    q_star = jnp.zeros((_B, 2 * _F), jnp.float32)
    for _ in range(3):
        gates = (q_star @ wih_ref[...] + bih_ref[...]
                 + h @ whh_ref[...] + bhh_ref[...])
        i_g, f_g, g_g, o_g = jnp.split(gates, 4, axis=-1)
        c = jax.nn.sigmoid(f_g) * c + jax.nn.sigmoid(i_g) * jnp.tanh(g_g)
        h = jax.nn.sigmoid(o_g) * jnp.tanh(c)
        q = h
        qn = sel @ q
        e = jnp.sum(x * qn, axis=-1, keepdims=True)
        emax = jnp.max(sel * e + (sel - 1.0) * _BIG, axis=0, keepdims=True)
        emax = jnp.where(emax > -_BIG * 0.5, emax, 0.0)
        a = jnp.exp(e - jnp.sum(sel * emax, axis=-1, keepdims=True))
        denom = jnp.sum(sel * a, axis=0, keepdims=True)
        denom_n = jnp.sum(sel * denom, axis=-1, keepdims=True)
        a = a / jnp.maximum(denom_n, 1e-16)
        r = jax.lax.dot_general(sel * a, x, (((0,), (0,)), ((), ())))
        q_star = jnp.concatenate([q, r], axis=-1)
    o1 = jnp.maximum(q_star @ w1_ref[...] + tt_ref[...] @ wt_ref[...]
                     + pp_ref[...] @ wp_ref[...] + b1_ref[...], 0.0)
    o2 = jnp.maximum(o1 @ w2_ref[...] + b2_ref[...], 0.0)
    out_ref[...] = o2 @ w3_ref[...] + b3_ref[...]


def _row_bs(shape):
    return pl.BlockSpec((_R,) + shape[1:], lambda i: (0,) * len(shape))


def _full_bs(shape):
    return pl.BlockSpec(shape, lambda i: (0,) * len(shape))


def _node_call(body, in_shapes, n_row_in, n_row_out, extra_out=()):
    grid = _N // _R
    in_specs = []
    for k, s in enumerate(in_shapes):
        if k < n_row_in:
            in_specs.append(pl.BlockSpec(
                (_R,) + tuple(s[1:]),
                lambda i, r=len(s): (i,) + (0,) * (r - 1)))
        else:
            in_specs.append(pl.BlockSpec(
                tuple(s), lambda i, r=len(s): (0,) * r))
    out_specs = [pl.BlockSpec((_R, _F), lambda i: (i, 0))
                 for _ in range(n_row_out)]
    out_shape = [jax.ShapeDtypeStruct((_N, _F), jnp.float32)
                 for _ in range(n_row_out)]
    for s in extra_out:
        out_specs.append(pl.BlockSpec(s, lambda i: (0,) * len(s)))
        out_shape.append(jax.ShapeDtypeStruct(s, jnp.float32))
    return pl.pallas_call(body, grid=(grid,), in_specs=in_specs,
                          out_specs=out_specs, out_shape=out_shape)


def _vmem_call(body, out_shape):
    return pl.pallas_call(body, out_shape=out_shape)


def kernel(x, edge_index, edge_attr, batch, t, p, params):
    src = edge_index[0]
    dst = edge_index[1]
    i32 = jnp.int32

    eids = jnp.arange(_E, dtype=i32)
    dst_sorted, perm = jax.lax.sort_key_val(dst, eids)
    pad1 = jnp.zeros((_EPAD - _E,), i32)
    dst_pad = jnp.concatenate([dst_sorted, pad1 + _N])
    eid_pad = jnp.concatenate([perm, pad1])

    attr8 = jnp.concatenate(
        [edge_attr, jnp.zeros((_E, 8 - edge_attr.shape[1]), jnp.float32)],
        axis=1)
    cmat = _make_cmat_call()
    seg = _make_sc_seg_call()

    W1, b1 = params['lin1']
    convs = params['convs']

    f32 = jnp.float32
    pre = _node_call(
        _pre_body,
        [(_N, _F), (_F, _F), (1, _F), (_F, _F), (_F, _F), (1, _F)],
        n_row_in=1, n_row_out=3)
    ab = _node_call(
        _ab_body,
        [(_N, _F), (1, _F), (1, _F), (_F, _F), (_F, _F), (1, _F)],
        n_row_in=1, n_row_out=3)
    post = _node_call(
        _post_body,
        [(_N, _F)] * 6 + [(_N, 1)] + [(_F, _F), (4 * _F, _F), (4 * _F, _F),
                                      (4 * _F, _F), (1, _F), (_F, _F),
                                      (1, _F)],
        n_row_in=7, n_row_out=1, extra_out=[(1, _F), (1, _F)])

    h = a = bm = None
    z = cs = csq = None
    deg = None
    for li, cp in enumerate(convs):
        wd = cp['Wpre'][:_F]
        ws = cp['Wpre'][_F:2 * _F]
        wpe = cp['Wpre'][2 * _F:]
        bpre = cp['bpre'].reshape(1, _F)
        if li == 0:
            h, a, bm = pre(x, W1, b1.reshape(1, _F), wd, ws, bpre)
        else:
            mean_c = cs / _N
            var_c = csq / _N - mean_c * mean_c
            s_aff = cp_prev['gamma'].reshape(1, _F) / jnp.sqrt(var_c + 1e-5)
            t_aff = cp_prev['beta'].reshape(1, _F) - mean_c * s_aff
            h, a, bm = ab(z, s_aff, t_aff, wd, ws, bpre)
        we8 = jnp.concatenate(
            [cp['We'], jnp.zeros((2, _F), jnp.float32)], axis=0)
        c_edges = cmat(attr8, we8, wpe, cp['be'].reshape(1, _F))
        s1p, sminp, smaxp, s2p, degp = seg(bm, c_edges, src, eid_pad,
                                           dst_pad)
        s1 = s1p.reshape(_NP, _F)[:_N]
        smin = sminp.reshape(_NP, _F)[:_N]
        smax = smaxp.reshape(_NP, _F)[:_N]
        s2 = s2p.reshape(_NP, _F)[:_N]
        if deg is None:
            deg = degp.reshape(_NP, 16)[:_N, :1]
        wh = cp['Wpost'][:_F]
        wa = cp['Wpost'][_F:5 * _F]
        wamp = cp['Wpost'][5 * _F:9 * _F]
        watt = cp['Wpost'][9 * _F:]
        z, cs, csq = post(h, a, s1, smin, smax, s2, deg, wh, wa, wamp, watt,
                          cp['bpost'].reshape(1, _F), cp['Wlin'],
                          cp['blin'].reshape(1, _F))
        cp_prev = cp

    mean_c = cs / _N
    var_c = csq / _N - mean_c * mean_c
    s_aff = cp_prev['gamma'].reshape(1, _F) / jnp.sqrt(var_c + 1e-5)
    t_aff = cp_prev['beta'].reshape(1, _F) - mean_c * s_aff

    lstm = params['lstm']
    (W1m, b1m), (W2m, b2m), (W3m, b3m) = params['mlp']
    s2s = _vmem_call(_s2s_body, jax.ShapeDtypeStruct((_B, 1), f32))
    out = s2s(z, batch.reshape(_N, 1), s_aff, t_aff,
              lstm['W_ih'].T, lstm['W_hh'].T,
              lstm['b_ih'].reshape(1, 4 * _F), lstm['b_hh'].reshape(1, 4 * _F),
              W1m[:2 * _F], W1m[2 * _F:2 * _F + 1], W1m[2 * _F + 1:],
              b1m.reshape(1, 64), W2m, b2m.reshape(1, 32), W3m,
              b3m.reshape(1, 1), t, p)
    return out.reshape(-1)


# speedup vs baseline: 1.9958x; 1.0489x over previous
import functools
import numpy as np
import jax
import jax.numpy as jnp
from jax import lax
from jax.experimental import pallas as pl
from jax.experimental.pallas import tpu as pltpu
from jax.experimental.pallas import tpu_sc as plsc

_N = 10000
_E = 320000
_F = 128
_B = 128
_AVG_DEG_LOG = float(np.log(33.0))
_BIG = 1e30

_K = 80
_SUP = 8
_SUPE = _K * _SUP
_RN = 160
_NRANGE = 64
_NP = _RN * _NRANGE
_EPAD = _E + 1024


def _sc_seg_body(bm_hbm, cs_hbm, src_hbm, eid_hbm, dst_hbm,
             s1_hbm, smin_hbm, smax_hbm, s2_hbm, deg_hbm,
             acc1, accmin, accmax, acc2, degacc,
             sidx, eidx, dstv, browA, crowA, browB, crowB, probev,
             semA, semB, semS):
    w = lax.axis_index("s") * 2 + lax.axis_index("c")
    zero16 = jnp.zeros((16,), jnp.float32)

    def init_regs():
        return tuple([zero16] * 8 + [zero16 + 3e38] * 8
                     + [zero16 - 3e38] * 8 + [zero16] * 8 + [zero16])

    def lower_bound(v):
        def it(t, lohi):
            lo, hi = lohi
            mid = (lo + hi) // 2
            m8 = (mid // 8) * 8
            pltpu.sync_copy(dst_hbm.at[pl.ds(m8, 16)],
                            probev.at[pl.ds(0, 16)])
            dv = probev[pl.ds(mid - m8, 16)][0]
            less = dv < v
            return (jnp.where(less, mid + 1, lo), jnp.where(less, hi, mid))
        lo, hi = lax.fori_loop(0, 19, it, (jnp.int32(0), jnp.int32(_E)),
                               unroll=False)
        return lo

    eb0 = lower_bound((w * 2) * _RN)
    eb1 = lower_bound((w * 2 + 1) * _RN)
    eb2 = lower_bound((w * 2 + 2) * _RN)

    for p in range(2):
        r = w * 2 + p
        n0 = r * _RN
        e0 = eb0 if p == 0 else eb1
        e1 = eb1 if p == 0 else eb2
        base0 = (e0 // 8) * 8
        nsuper = (e1 - base0 + _SUPE - 1) // _SUPE

        def zbody(i, _):
            for c in range(8):
                sl = pl.ds(i * _F + c * 16, 16)
                acc1[sl] = zero16
                accmin[sl] = zero16
                accmax[sl] = zero16
                acc2[sl] = zero16
            degacc[pl.ds(i * 16, 16)] = zero16
            return 0
        lax.fori_loop(0, _RN, zbody, 0, unroll=False)

        def flush(d, regs):
            for c in range(8):
                sl = pl.ds(d * _F + c * 16, 16)
                acc1[sl] = regs[c]
                accmin[sl] = regs[8 + c]
                accmax[sl] = regs[16 + c]
                acc2[sl] = regs[24 + c]
            degacc[pl.ds(d * 16, 16)] = regs[32]

        bufs = ((browA, crowA, semA), (browB, crowB, semB))

        def gathers(c, buf):
            brow, crow, sem = buf
            cp1 = pltpu.async_copy(bm_hbm.at[sidx.at[pl.ds(c * _K, _K)]],
                                   brow, sem)
            cp2 = pltpu.async_copy(cs_hbm.at[eidx.at[pl.ds(c * _K, _K)]],
                                   crow, sem)
            return cp1, cp2

        def consume(carry, brow, crow, sstart, c):
            cstart = sstart + c * _K
            lo = jnp.maximum(e0 - cstart, 0)
            hi = jnp.minimum(e1 - cstart, _K)

            def edge_body(e, st):
                d_prev = st[0]
                d = dstv[pl.ds(c * _K + e, 16)][0] - n0
                pred = d != d_prev

                @pl.when(jnp.logical_and(pred, d_prev >= 0))
                def _():
                    flush(d_prev, st[1:])

                new = [d]
                for cc in range(8):
                    sl = pl.ds(cc * 16, 16)
                    u = brow[e, sl] + crow[e, sl]
                    u2 = u * u
                    new.append(jnp.where(pred, u, st[1 + cc] + u))
                    new.append(jnp.where(pred, u, jnp.minimum(st[9 + cc], u)))
                    new.append(jnp.where(pred, u, jnp.maximum(st[17 + cc], u)))
                    new.append(jnp.where(pred, u2, st[25 + cc] + u2))
                out = [new[0]]
                out += [new[1 + 4 * cc] for cc in range(8)]
                out += [new[2 + 4 * cc] for cc in range(8)]
                out += [new[3 + 4 * cc] for cc in range(8)]
                out += [new[4 + 4 * cc] for cc in range(8)]
                out.append(jnp.where(pred, zero16 + 1.0, st[33] + 1.0))
                return tuple(out)
            return lax.fori_loop(lo, hi, edge_body, carry, unroll=False)

        def super_body(s, carry):
            sstart = base0 + s * _SUPE
            pltpu.sync_copy(eid_hbm.at[pl.ds(sstart, _SUPE)], eidx)
            pltpu.sync_copy(dst_hbm.at[pl.ds(sstart, _SUPE)],
                            dstv.at[pl.ds(0, _SUPE)])
            scp = [pltpu.async_copy(
                src_hbm.at[eidx.at[pl.ds(j * _K, _K)]],
                sidx.at[pl.ds(j * _K, _K)], semS) for j in range(_SUP)]
            for cp in scp:
                cp.wait()
            pend = gathers(0, bufs[0])
            for c in range(_SUP):
                nxt = gathers(c + 1, bufs[(c + 1) % 2]) if c + 1 < _SUP else None
                pend[0].wait()
                pend[1].wait()
                brow, crow, _ = bufs[c % 2]
                carry = consume(carry, brow, crow, sstart, c)
                pend = nxt
            return carry

        carry = (jnp.int32(-1),) + init_regs()
        carry = lax.fori_loop(0, nsuper, super_body, carry, unroll=False)

        d_last = carry[0]

        @pl.when(d_last >= 0)
        def _():
            flush(d_last, carry[1:])

        pltpu.sync_copy(acc1, s1_hbm.at[pl.ds(n0 * _F, _RN * _F)])
        pltpu.sync_copy(accmin, smin_hbm.at[pl.ds(n0 * _F, _RN * _F)])
        pltpu.sync_copy(accmax, smax_hbm.at[pl.ds(n0 * _F, _RN * _F)])
        pltpu.sync_copy(acc2, s2_hbm.at[pl.ds(n0 * _F, _RN * _F)])
        pltpu.sync_copy(degacc, deg_hbm.at[pl.ds(n0 * 16, _RN * 16)])


def _make_sc_seg_call():
    f32 = jnp.float32
    i32 = jnp.int32
    mesh = plsc.VectorSubcoreMesh(core_axis_name="c", subcore_axis_name="s")
    return pl.kernel(
        _sc_seg_body,
        out_type=[jax.ShapeDtypeStruct((_NP * _F,), f32) for _ in range(4)]
        + [jax.ShapeDtypeStruct((_NP * 16,), f32)],
        mesh=mesh,
        scratch_types=[
            pltpu.VMEM((_RN * _F,), f32), pltpu.VMEM((_RN * _F,), f32),
            pltpu.VMEM((_RN * _F,), f32), pltpu.VMEM((_RN * _F,), f32),
            pltpu.VMEM((_RN * 16,), f32),
            pltpu.VMEM((_SUPE,), i32), pltpu.VMEM((_SUPE,), i32),
            pltpu.VMEM((_SUPE + 16,), i32),
            pltpu.VMEM((_K, _F), f32), pltpu.VMEM((_K, _F), f32),
            pltpu.VMEM((_K, _F), f32), pltpu.VMEM((_K, _F), f32),
            pltpu.VMEM((32,), i32),
            pltpu.SemaphoreType.DMA, pltpu.SemaphoreType.DMA,
            pltpu.SemaphoreType.DMA,
        ],
    )


_CBLK = 4000


def _cmat_body(attr_ref, we_ref, wpe_ref, be_ref, c_ref):
    wfold = we_ref[...] @ wpe_ref[...]
    bfold = be_ref[...] @ wpe_ref[...]
    c_ref[...] = attr_ref[...] @ wfold + bfold


def _make_cmat_call():
    f32 = jnp.float32
    return pl.pallas_call(
        _cmat_body,
        grid=(_E // _CBLK,),
        in_specs=[
            pl.BlockSpec((_CBLK, 8), lambda i: (i, 0)),
            pl.BlockSpec((8, _F), lambda i: (0, 0)),
            pl.BlockSpec((_F, _F), lambda i: (0, 0)),
            pl.BlockSpec((1, _F), lambda i: (0, 0)),
        ],
        out_specs=pl.BlockSpec((_CBLK, _F), lambda i: (i, 0)),
        out_shape=jax.ShapeDtypeStruct((_E, _F), f32),
    )


_R = 2000


def _pre_body(x_ref, w1_ref, b1_ref, wd_ref, ws_ref, bpre_ref,
              h_ref, a_ref, bm_ref):
    h = jnp.maximum(x_ref[...] @ w1_ref[...] + b1_ref[...], 0.0)
    h_ref[...] = h
    a_ref[...] = h @ wd_ref[...] + bpre_ref[...]
    bm_ref[...] = h @ ws_ref[...]


def _ab_body(z_ref, s_ref, t_ref, wd_ref, ws_ref, bpre_ref,
             h_ref, a_ref, bm_ref):
    h = jnp.maximum(z_ref[...] * s_ref[...] + t_ref[...], 0.0)
    h_ref[...] = h
    a_ref[...] = h @ wd_ref[...] + bpre_ref[...]
    bm_ref[...] = h @ ws_ref[...]


def _post_body(h_ref, a_ref, s1_ref, smin_ref, smax_ref, s2_ref, deg_ref,
               wh_ref, wa_ref, wamp_ref, watt_ref, bpost_ref, wlin_ref,
               blin_ref, z_ref, cs_ref, csq_ref):
    deg = deg_ref[...]
    degc = jnp.maximum(deg, 1.0)
    a = a_ref[...]
    s1 = s1_ref[...].reshape(_R, _F)
    inv = 1.0 / degc
    has = deg > 0.0
    mu = s1 * inv
    mean = a * (deg * inv) + mu
    mn = jnp.where(has, a + smin_ref[...].reshape(_R, _F), 0.0)
    mx = jnp.where(has, a + smax_ref[...].reshape(_R, _F), 0.0)
    var = s2_ref[...].reshape(_R, _F) * inv - mu * mu
    std = jnp.sqrt(jnp.maximum(var, 0.0) + 1e-5)
    agg = jnp.concatenate([mean, mn, mx, std], axis=-1)
    lg = jnp.log(degc + 1.0)
    amp = lg * (1.0 / _AVG_DEG_LOG)
    att = _AVG_DEG_LOG / lg
    y = (h_ref[...] @ wh_ref[...] + agg @ wa_ref[...]
         + amp * (agg @ wamp_ref[...]) + att * (agg @ watt_ref[...])
         + bpost_ref[...])
    z = y @ wlin_ref[...] + blin_ref[...]
    z_ref[...] = z

    @pl.when(pl.program_id(0) == 0)
    def _init():
        cs_ref[...] = jnp.zeros_like(cs_ref)
        csq_ref[...] = jnp.zeros_like(csq_ref)

    cs_ref[...] += jnp.sum(z, axis=0, keepdims=True)
    csq_ref[...] += jnp.sum(z * z, axis=0, keepdims=True)


def _s2s_body(x_ref, bt_ref, s_aff_ref, t_aff_ref, wih_ref, whh_ref, bih_ref,
              bhh_ref, w1_ref, wt_ref, wp_ref, b1_ref, w2_ref, b2_ref,
              w3_ref, b3_ref, tt_ref, pp_ref, out_ref):
    x = jnp.maximum(x_ref[...] * s_aff_ref[...] + t_aff_ref[...], 0.0)
    bt = bt_ref[...]
    cols = jax.lax.broadcasted_iota(jnp.int32, (_N, _B), 1)
    sel = (bt == cols).astype(jnp.float32)
    h = jnp.zeros((_B, _F), jnp.float32)
    c = jnp.zeros((_B, _F), jnp.float32)
    q_star = jnp.zeros((_B, 2 * _F), jnp.float32)
    for _ in range(3):
        gates = (q_star @ wih_ref[...] + bih_ref[...]
                 + h @ whh_ref[...] + bhh_ref[...])
        i_g, f_g, g_g, o_g = jnp.split(gates, 4, axis=-1)
        c = jax.nn.sigmoid(f_g) * c + jax.nn.sigmoid(i_g) * jnp.tanh(g_g)
        h = jax.nn.sigmoid(o_g) * jnp.tanh(c)
        q = h
        qn = sel @ q
        e = jnp.sum(x * qn, axis=-1, keepdims=True)
        emax = jnp.max(sel * e + (sel - 1.0) * _BIG, axis=0, keepdims=True)
        emax = jnp.where(emax > -_BIG * 0.5, emax, 0.0)
        a = jnp.exp(e - jnp.sum(sel * emax, axis=-1, keepdims=True))
        denom = jnp.sum(sel * a, axis=0, keepdims=True)
        denom_n = jnp.sum(sel * denom, axis=-1, keepdims=True)
        a = a / jnp.maximum(denom_n, 1e-16)
        r = jax.lax.dot_general(sel * a, x, (((0,), (0,)), ((), ())))
        q_star = jnp.concatenate([q, r], axis=-1)
    o1 = jnp.maximum(q_star @ w1_ref[...] + tt_ref[...] @ wt_ref[...]
                     + pp_ref[...] @ wp_ref[...] + b1_ref[...], 0.0)
    o2 = jnp.maximum(o1 @ w2_ref[...] + b2_ref[...], 0.0)
    out_ref[...] = o2 @ w3_ref[...] + b3_ref[...]


def _row_bs(shape):
    return pl.BlockSpec((_R,) + shape[1:], lambda i: (0,) * len(shape))


def _full_bs(shape):
    return pl.BlockSpec(shape, lambda i: (0,) * len(shape))


def _node_call(body, in_shapes, n_row_in, n_row_out, extra_out=()):
    grid = _N // _R
    in_specs = []
    for k, s in enumerate(in_shapes):
        if k < n_row_in:
            in_specs.append(pl.BlockSpec(
                (_R,) + tuple(s[1:]),
                lambda i, r=len(s): (i,) + (0,) * (r - 1)))
        else:
            in_specs.append(pl.BlockSpec(
                tuple(s), lambda i, r=len(s): (0,) * r))
    out_specs = [pl.BlockSpec((_R, _F), lambda i: (i, 0))
                 for _ in range(n_row_out)]
    out_shape = [jax.ShapeDtypeStruct((_N, _F), jnp.float32)
                 for _ in range(n_row_out)]
    for s in extra_out:
        out_specs.append(pl.BlockSpec(s, lambda i: (0,) * len(s)))
        out_shape.append(jax.ShapeDtypeStruct(s, jnp.float32))
    return pl.pallas_call(body, grid=(grid,), in_specs=in_specs,
                          out_specs=out_specs, out_shape=out_shape)


def _vmem_call(body, out_shape):
    return pl.pallas_call(body, out_shape=out_shape)


def kernel(x, edge_index, edge_attr, batch, t, p, params):
    src = edge_index[0]
    dst = edge_index[1]
    i32 = jnp.int32

    eids = jnp.arange(_E, dtype=i32)
    dst_sorted, perm = jax.lax.sort_key_val(dst, eids, is_stable=False)
    pad1 = jnp.zeros((_EPAD - _E,), i32)
    dst_pad = jnp.concatenate([dst_sorted, pad1 + _N])
    eid_pad = jnp.concatenate([perm, pad1])

    attr8 = jnp.concatenate(
        [edge_attr, jnp.zeros((_E, 8 - edge_attr.shape[1]), jnp.float32)],
        axis=1)
    cmat = _make_cmat_call()
    seg = _make_sc_seg_call()

    W1, b1 = params['lin1']
    convs = params['convs']

    f32 = jnp.float32
    pre = _node_call(
        _pre_body,
        [(_N, _F), (_F, _F), (1, _F), (_F, _F), (_F, _F), (1, _F)],
        n_row_in=1, n_row_out=3)
    ab = _node_call(
        _ab_body,
        [(_N, _F), (1, _F), (1, _F), (_F, _F), (_F, _F), (1, _F)],
        n_row_in=1, n_row_out=3)
    post_in_specs = [
        pl.BlockSpec((_R, _F), lambda i: (i, 0)),
        pl.BlockSpec((_R, _F), lambda i: (i, 0)),
        pl.BlockSpec((_R * _F,), lambda i: (i,)),
        pl.BlockSpec((_R * _F,), lambda i: (i,)),
        pl.BlockSpec((_R * _F,), lambda i: (i,)),
        pl.BlockSpec((_R * _F,), lambda i: (i,)),
        pl.BlockSpec((_R, 1), lambda i: (i, 0)),
        pl.BlockSpec((_F, _F), lambda i: (0, 0)),
        pl.BlockSpec((4 * _F, _F), lambda i: (0, 0)),
        pl.BlockSpec((4 * _F, _F), lambda i: (0, 0)),
        pl.BlockSpec((4 * _F, _F), lambda i: (0, 0)),
        pl.BlockSpec((1, _F), lambda i: (0, 0)),
        pl.BlockSpec((_F, _F), lambda i: (0, 0)),
        pl.BlockSpec((1, _F), lambda i: (0, 0)),
    ]
    post = pl.pallas_call(
        _post_body,
        grid=(_N // _R,),
        in_specs=post_in_specs,
        out_specs=[pl.BlockSpec((_R, _F), lambda i: (i, 0)),
                   pl.BlockSpec((1, _F), lambda i: (0, 0)),
                   pl.BlockSpec((1, _F), lambda i: (0, 0))],
        out_shape=[jax.ShapeDtypeStruct((_N, _F), jnp.float32),
                   jax.ShapeDtypeStruct((1, _F), jnp.float32),
                   jax.ShapeDtypeStruct((1, _F), jnp.float32)])

    h = a = bm = None
    z = cs = csq = None
    deg = None
    for li, cp in enumerate(convs):
        wd = cp['Wpre'][:_F]
        ws = cp['Wpre'][_F:2 * _F]
        wpe = cp['Wpre'][2 * _F:]
        bpre = cp['bpre'].reshape(1, _F)
        if li == 0:
            h, a, bm = pre(x, W1, b1.reshape(1, _F), wd, ws, bpre)
        else:
            mean_c = cs / _N
            var_c = csq / _N - mean_c * mean_c
            s_aff = cp_prev['gamma'].reshape(1, _F) / jnp.sqrt(var_c + 1e-5)
            t_aff = cp_prev['beta'].reshape(1, _F) - mean_c * s_aff
            h, a, bm = ab(z, s_aff, t_aff, wd, ws, bpre)
        we8 = jnp.concatenate(
            [cp['We'], jnp.zeros((2, _F), jnp.float32)], axis=0)
        c_edges = cmat(attr8, we8, wpe, cp['be'].reshape(1, _F))
        s1p, sminp, smaxp, s2p, degp = seg(bm, c_edges, src, eid_pad,
                                           dst_pad)
        if deg is None:
            deg = degp.reshape(_NP, 16)[:_N, :1]
        wh = cp['Wpost'][:_F]
        wa = cp['Wpost'][_F:5 * _F]
        wamp = cp['Wpost'][5 * _F:9 * _F]
        watt = cp['Wpost'][9 * _F:]
        z, cs, csq = post(h, a, s1p, sminp, smaxp, s2p, deg, wh, wa, wamp,
                          watt, cp['bpost'].reshape(1, _F), cp['Wlin'],
                          cp['blin'].reshape(1, _F))
        cp_prev = cp

    mean_c = cs / _N
    var_c = csq / _N - mean_c * mean_c
    s_aff = cp_prev['gamma'].reshape(1, _F) / jnp.sqrt(var_c + 1e-5)
    t_aff = cp_prev['beta'].reshape(1, _F) - mean_c * s_aff

    lstm = params['lstm']
    (W1m, b1m), (W2m, b2m), (W3m, b3m) = params['mlp']
    s2s = _vmem_call(_s2s_body, jax.ShapeDtypeStruct((_B, 1), f32))
    out = s2s(z, batch.reshape(_N, 1), s_aff, t_aff,
              lstm['W_ih'].T, lstm['W_hh'].T,
              lstm['b_ih'].reshape(1, 4 * _F), lstm['b_hh'].reshape(1, 4 * _F),
              W1m[:2 * _F], W1m[2 * _F:2 * _F + 1], W1m[2 * _F + 1:],
              b1m.reshape(1, 64), W2m, b2m.reshape(1, 32), W3m,
              b3m.reshape(1, 1), t, p)
    return out.reshape(-1)
